# baseline (speedup 1.0000x reference)
import jax
import jax.numpy as jnp
from jax.experimental import pallas as pl
from jax.experimental.pallas import tpu as pltpu

N_CLASSES = 81
TOP_K = 200
CONF_THRESH = 0.01
IOU_THRESH = 0.45
IMG_H = 512.0
IMG_W = 512.0
CHUNK = 1000
SEL_R = 8
SEL_C = 32


def _body(y_ref, ocls_ref, oconf_ref, ox1_ref, oy1_ref, ox2_ref, oy2_ref,
          s_ref):
    n = y_ref.shape[1]
    nch = n // CHUNK

    def score_chunk(k, _):
        y = y_ref[0, pl.ds(k * CHUNK, CHUNK), :]
        ycls = y[:, :N_CLASSES]
        conf = jnp.max(ycls, axis=1)
        y0 = y[:, 0]
        valid = (y0 < conf) & (conf > CONF_THRESH)
        s_ref[pl.ds(k, 1), :] = jnp.where(valid, conf, -1.0).reshape(1, CHUNK)
        return 0

    jax.lax.fori_loop(0, nch, score_chunk, 0)

    zrow = jnp.zeros((TOP_K, 1), jnp.float32)
    ocls_ref[0, :, :] = zrow
    oconf_ref[0, :, :] = zrow
    ox1_ref[0, :, :] = zrow
    oy1_ref[0, :, :] = zrow
    ox2_ref[0, :, :] = zrow
    oy2_ref[0, :, :] = zrow

    fiota = (jax.lax.broadcasted_iota(jnp.int32, (nch, CHUNK), 0) * CHUNK
             + jax.lax.broadcasted_iota(jnp.int32, (nch, CHUNK), 1))
    lane_c = jax.lax.broadcasted_iota(jnp.int32, (1, CHUNK), 1)
    sel_io = (jax.lax.broadcasted_iota(jnp.int32, (SEL_R, SEL_C), 0) * SEL_C
              + jax.lax.broadcasted_iota(jnp.int32, (SEL_R, SEL_C), 1))
    cls_io = jax.lax.broadcasted_iota(jnp.int32, (1, N_CLASSES), 1)

    def argmax_s():
        s = s_ref[:, :]
        m = jnp.max(s)
        idx = jnp.min(jnp.where(s == m, fiota, n))
        return m, idx

    zsel = jnp.zeros((SEL_R, SEL_C), jnp.float32)
    m0, i0 = argmax_s()

    def cond(carry):
        nsel, m = carry[0], carry[1]
        return (nsel < TOP_K) & (m > 0.0)

    def body(carry):
        nsel, m, idx, ex1, ey1, ex2, ey2, ear = carry
        r = idx // CHUNK
        c = idx - r * CHUNK

        yrow = y_ref[0, pl.ds(idx, 1), :]
        clsv = jnp.min(jnp.where(yrow[:, :N_CLASSES] == m, cls_io,
                                 N_CLASSES))
        scl = clsv.astype(jnp.float32)
        cxv = (yrow[:, 81:82] * yrow[:, 89:90] * yrow[:, 87:88]
               + yrow[:, 85:86])
        cyv = (yrow[:, 82:83] * yrow[:, 90:91] * yrow[:, 88:89]
               + yrow[:, 86:87])
        ev = jnp.exp(yrow[:, 83:85] * yrow[:, 91:93])
        wv = ev[:, 0:1] * yrow[:, 87:88]
        hv = ev[:, 1:2] * yrow[:, 88:89]
        sx1 = jnp.sum((cxv - 0.5 * wv) * IMG_W)
        sy1 = jnp.sum((cyv - 0.5 * hv) * IMG_H)
        sx2 = jnp.sum((cxv + 0.5 * wv) * IMG_W)
        sy2 = jnp.sum((cyv + 0.5 * hv) * IMG_H)
        sar = (jnp.maximum(sx2 - sx1, 0.0) * jnp.maximum(sy2 - sy1, 0.0))

        ix1 = jnp.maximum(ex1, sx1)
        iy1 = jnp.maximum(ey1, sy1)
        ix2 = jnp.minimum(ex2, sx2)
        iy2 = jnp.minimum(ey2, sy2)
        inter = jnp.maximum(ix2 - ix1, 0.0) * jnp.maximum(iy2 - iy1, 0.0)
        union = jnp.maximum(ear + sar - inter, 1e-9)
        supp = jnp.any(inter / union > IOU_THRESH)
        acc = jnp.logical_not(supp)
        accf = acc.astype(jnp.float32)

        oh = (sel_io == nsel) & acc
        ex1n = jnp.where(oh, sx1, ex1)
        ey1n = jnp.where(oh, sy1, ey1)
        ex2n = jnp.where(oh, sx2, ex2)
        ey2n = jnp.where(oh, sy2, ey2)
        earn = jnp.where(oh, sar, ear)

        ocls_ref[0, pl.ds(nsel, 1), :] = (accf * scl).reshape(1, 1)
        oconf_ref[0, pl.ds(nsel, 1), :] = (accf * m).reshape(1, 1)
        ox1_ref[0, pl.ds(nsel, 1), :] = (accf * sx1).reshape(1, 1)
        oy1_ref[0, pl.ds(nsel, 1), :] = (accf * sy1).reshape(1, 1)
        ox2_ref[0, pl.ds(nsel, 1), :] = (accf * sx2).reshape(1, 1)
        oy2_ref[0, pl.ds(nsel, 1), :] = (accf * sy2).reshape(1, 1)

        srow = s_ref[pl.ds(r, 1), :]
        s_ref[pl.ds(r, 1), :] = jnp.where(lane_c == c, -1.0, srow)

        m2, i2 = argmax_s()
        return (nsel + acc.astype(jnp.int32), m2, i2,
                ex1n, ey1n, ex2n, ey2n, earn)

    jax.lax.while_loop(cond, body,
                       (jnp.int32(0), m0, i0, zsel, zsel, zsel, zsel, zsel))


def kernel(y_pred):
    b, n, c = y_pred.shape
    nch = n // CHUNK
    out_sds = jax.ShapeDtypeStruct((b, TOP_K, 1), jnp.float32)
    out_spec = pl.BlockSpec((1, TOP_K, 1), lambda i: (i, 0, 0))
    outs = pl.pallas_call(
        _body,
        grid=(b,),
        in_specs=[pl.BlockSpec((1, n, c), lambda i: (i, 0, 0))],
        out_specs=[out_spec] * 6,
        out_shape=[out_sds] * 6,
        scratch_shapes=[pltpu.VMEM((nch, CHUNK), jnp.float32)],
        compiler_params=pltpu.CompilerParams(
            dimension_semantics=("parallel",)),
    )(y_pred)
    cls, conf, x1, y1, x2, y2 = [o[..., 0] for o in outs]
    return jnp.stack([cls, conf, x1, y1, x2, y2], axis=-1)
